# baseline (speedup 1.0000x reference)
import jax
import jax.numpy as jnp
from jax import lax
from jax.experimental import pallas as pl
from jax.experimental.pallas import tpu as pltpu
from jax.experimental.pallas import tpu_sc as plsc

_B = 512
_BS = 19
_P = _BS * _BS
_MAXM = _P
_EMPTY = -1.0
_NW = 32
_GPW = _B // _NW
_LANES = 16
_PPAD = 368
_NCHUNK = _PPAD // _LANES
_GB = 8


_K = 16


def _tc_scat_body(mv_ref, board_ref, hist_in, hist_out, sems):
    del hist_in

    def _desc(i):
        row = i * _MAXM + mv_ref[i]
        return pltpu.make_async_copy(board_ref.at[i], hist_out.at[row],
                                     sems.at[lax.rem(i, _K)])

    def body(i, carry):
        _desc(i).start()

        @pl.when(i >= _K)
        def _():
            _desc(i - _K).wait()

        return carry

    lax.fori_loop(0, _B, body, jnp.int32(0))

    def drain(j, carry):
        _desc(_B - _K + j).wait()
        return carry

    lax.fori_loop(0, _K, drain, jnp.int32(0))


_tc_scat = pl.pallas_call(
    _tc_scat_body,
    in_specs=[
        pl.BlockSpec(memory_space=pltpu.SMEM),
        pl.BlockSpec(memory_space=pltpu.VMEM),
        pl.BlockSpec(memory_space=pltpu.HBM),
    ],
    out_specs=pl.BlockSpec(memory_space=pltpu.HBM),
    out_shape=jax.ShapeDtypeStruct((_B * _MAXM, _P), jnp.float32),
    input_output_aliases={2: 0},
    scratch_shapes=[pltpu.SemaphoreType.DMA((_K,))],
    name="go_hist_scatter_tc",
)


def _sc_board_body(bpad_h, roots_h, rows_h, cols_h, ply_h, cg_h,
                   board_out,
                   bpad_v, roots_v, rows_v, cols_v, ply_v, cg_v,
                   pos_s, play_s, plyf_s):
    wid = lax.axis_index("s") * 2 + lax.axis_index("c")
    base = wid * _GPW

    pltpu.sync_copy(rows_h.at[pl.ds(base, _GPW)], rows_v)
    pltpu.sync_copy(cols_h.at[pl.ds(base, _GPW)], cols_v)
    pltpu.sync_copy(ply_h.at[pl.ds(base, _GPW)], ply_v)
    pltpu.sync_copy(bpad_h.at[pl.ds(base, _GPW)], bpad_v)
    pltpu.sync_copy(roots_h.at[pl.ds(base, _GPW)], roots_v)
    pltpu.sync_copy(cg_h.at[pl.ds(base * _P * 4, _GPW * _P * 4)], cg_v)

    iota = lax.iota(jnp.int32, _LANES)
    rv = rows_v[...]
    cv = cols_v[...]
    rc = jnp.clip(rv, 0, _BS - 1)
    cc = jnp.clip(cv, 0, _BS - 1)
    posv = rc * _BS + cc
    playv = jnp.where((rv >= 0) & (cv >= 0), jnp.int32(1), jnp.int32(0))
    plyfv = ply_v[...].astype(jnp.float32)

    pos_s[...] = posv
    play_s[...] = playv
    plyf_s[...] = plyfv

    def game_body(i, carry):
        isplat = jnp.full((_LANES,), i, jnp.int32)
        poss = plsc.load_gather(pos_s, [isplat])
        plays = plsc.load_gather(play_s, [isplat]) != 0
        plysf = plsc.load_gather(plyf_s, [isplat])
        cgbase = jnp.full((_LANES,), i * (_P * 4), jnp.int32) + poss * 4
        g0 = plsc.load_gather(cg_v, [cgbase])
        g1 = plsc.load_gather(cg_v, [cgbase + 1])
        g2 = plsc.load_gather(cg_v, [cgbase + 2])
        g3 = plsc.load_gather(cg_v, [cgbase + 3])

        def chunk_body(j, c2):
            off = j * _LANES
            lanes = off + iota
            bvals = bpad_v[i, pl.ds(off, _LANES)]
            rvals = roots_v[i, pl.ds(off, _LANES)]
            v = jnp.where(plays & (lanes == poss), plysf, bvals)
            cap = (((rvals == g0) & (g0 >= 0)) | ((rvals == g1) & (g1 >= 0))
                   | ((rvals == g2) & (g2 >= 0)) | ((rvals == g3) & (g3 >= 0)))
            v = jnp.where(plays & cap, jnp.float32(_EMPTY), v)
            bpad_v[i, pl.ds(off, _LANES)] = v
            return c2

        return lax.fori_loop(0, _NCHUNK, chunk_body, carry)

    lax.fori_loop(0, _GPW, game_body, jnp.int32(0))

    pltpu.sync_copy(bpad_v, board_out.at[pl.ds(base, _GPW)])


_mesh = plsc.VectorSubcoreMesh(core_axis_name="c", subcore_axis_name="s")

_sc_board = pl.kernel(
    _sc_board_body,
    out_type=jax.ShapeDtypeStruct((_B, _PPAD), jnp.float32),
    mesh=_mesh,
    scratch_types=[
        pltpu.VMEM((_GPW, _PPAD), jnp.float32),
        pltpu.VMEM((_GPW, _PPAD), jnp.int32),
        pltpu.VMEM((_GPW,), jnp.int32),
        pltpu.VMEM((_GPW,), jnp.int32),
        pltpu.VMEM((_GPW,), jnp.int32),
        pltpu.VMEM((_GPW * _P * 4,), jnp.int32),
        pltpu.VMEM((_GPW,), jnp.int32),
        pltpu.VMEM((_GPW,), jnp.int32),
        pltpu.VMEM((_GPW,), jnp.float32),
    ],
    compiler_params=pltpu.CompilerParams(needs_layout_passes=False),
    name="go_board_sc",
)


def kernel(board, board_history, positions, current_player, pass_count,
           move_count, roots, capture_groups):
    del pass_count
    board_flat = board.reshape(_B, _P)
    board_pad = jnp.pad(board_flat, ((0, 0), (0, _PPAD - _P)))
    roots_pad = jnp.pad(roots, ((0, 0), (0, _PPAD - _P)), constant_values=-1)
    rows = positions[:, 0]
    cols = positions[:, 1]
    cg1d = capture_groups.reshape(_B * _P * 4)
    hist2d = board_history.reshape(_B * _MAXM, _P)
    hist_out = _tc_scat(move_count, board_flat, hist2d)
    board_out = _sc_board(board_pad, roots_pad, rows, cols, current_player,
                          cg1d)
    new_board = board_out[:, :_P].reshape(_B, _BS, _BS)
    return new_board, hist_out.reshape(_B, _MAXM, _P)


# speedup vs baseline: 1.7157x; 1.7157x over previous
import jax
import jax.numpy as jnp
from jax import lax
from jax.experimental import pallas as pl
from jax.experimental.pallas import tpu as pltpu
from jax.experimental.pallas import tpu_sc as plsc

_B = 512
_BS = 19
_P = _BS * _BS
_MAXM = _P
_EMPTY = -1.0
_NW = 32
_GPW = _B // _NW
_LANES = 16
_PPAD = 368
_NCHUNK = _PPAD // _LANES
_GB = 8


_K = 16


def _tc_scat_body(mv_ref, board_ref, hist_in, hist_out, sems):
    del hist_in

    def _desc(i):
        return pltpu.make_async_copy(board_ref.at[i], hist_out.at[i, mv_ref[i]],
                                     sems.at[lax.rem(i, _K)])

    def body(i, carry):
        _desc(i).start()

        @pl.when(i >= _K)
        def _():
            _desc(i - _K).wait()

        return carry

    lax.fori_loop(0, _B, body, jnp.int32(0))

    def drain(j, carry):
        _desc(_B - _K + j).wait()
        return carry

    lax.fori_loop(0, _K, drain, jnp.int32(0))


_tc_scat = pl.pallas_call(
    _tc_scat_body,
    in_specs=[
        pl.BlockSpec(memory_space=pltpu.SMEM),
        pl.BlockSpec(memory_space=pltpu.VMEM),
        pl.BlockSpec(memory_space=pltpu.HBM),
    ],
    out_specs=pl.BlockSpec(memory_space=pltpu.HBM),
    out_shape=jax.ShapeDtypeStruct((_B, _MAXM, _P), jnp.float32),
    input_output_aliases={2: 0},
    scratch_shapes=[pltpu.SemaphoreType.DMA((_K,))],
    name="go_hist_scatter_tc",
)


def _sc_board_body(bpad_h, roots_h, rows_h, cols_h, ply_h, cg_h,
                   board_out,
                   bpad_v, roots_v, rows_v, cols_v, ply_v, cg_v,
                   pos_s, play_s, plyf_s):
    wid = lax.axis_index("s") * 2 + lax.axis_index("c")
    base = wid * _GPW

    pltpu.sync_copy(rows_h.at[pl.ds(base, _GPW)], rows_v)
    pltpu.sync_copy(cols_h.at[pl.ds(base, _GPW)], cols_v)
    pltpu.sync_copy(ply_h.at[pl.ds(base, _GPW)], ply_v)
    pltpu.sync_copy(bpad_h.at[pl.ds(base, _GPW)], bpad_v)
    pltpu.sync_copy(roots_h.at[pl.ds(base, _GPW)], roots_v)
    pltpu.sync_copy(cg_h.at[pl.ds(base * _P * 4, _GPW * _P * 4)], cg_v)

    iota = lax.iota(jnp.int32, _LANES)
    rv = rows_v[...]
    cv = cols_v[...]
    rc = jnp.clip(rv, 0, _BS - 1)
    cc = jnp.clip(cv, 0, _BS - 1)
    posv = rc * _BS + cc
    playv = jnp.where((rv >= 0) & (cv >= 0), jnp.int32(1), jnp.int32(0))
    plyfv = ply_v[...].astype(jnp.float32)

    pos_s[...] = posv
    play_s[...] = playv
    plyf_s[...] = plyfv

    def game_body(i, carry):
        isplat = jnp.full((_LANES,), i, jnp.int32)
        poss = plsc.load_gather(pos_s, [isplat])
        plays = plsc.load_gather(play_s, [isplat]) != 0
        plysf = plsc.load_gather(plyf_s, [isplat])
        cgbase = jnp.full((_LANES,), i * (_P * 4), jnp.int32) + poss * 4
        g0 = plsc.load_gather(cg_v, [cgbase])
        g1 = plsc.load_gather(cg_v, [cgbase + 1])
        g2 = plsc.load_gather(cg_v, [cgbase + 2])
        g3 = plsc.load_gather(cg_v, [cgbase + 3])

        def chunk_body(j, c2):
            off = j * _LANES
            lanes = off + iota
            bvals = bpad_v[i, pl.ds(off, _LANES)]
            rvals = roots_v[i, pl.ds(off, _LANES)]
            v = jnp.where(plays & (lanes == poss), plysf, bvals)
            cap = (((rvals == g0) & (g0 >= 0)) | ((rvals == g1) & (g1 >= 0))
                   | ((rvals == g2) & (g2 >= 0)) | ((rvals == g3) & (g3 >= 0)))
            v = jnp.where(plays & cap, jnp.float32(_EMPTY), v)
            bpad_v[i, pl.ds(off, _LANES)] = v
            return c2

        return lax.fori_loop(0, _NCHUNK, chunk_body, carry)

    lax.fori_loop(0, _GPW, game_body, jnp.int32(0))

    pltpu.sync_copy(bpad_v, board_out.at[pl.ds(base, _GPW)])


_mesh = plsc.VectorSubcoreMesh(core_axis_name="c", subcore_axis_name="s")

_sc_board = pl.kernel(
    _sc_board_body,
    out_type=jax.ShapeDtypeStruct((_B, _PPAD), jnp.float32),
    mesh=_mesh,
    scratch_types=[
        pltpu.VMEM((_GPW, _PPAD), jnp.float32),
        pltpu.VMEM((_GPW, _PPAD), jnp.int32),
        pltpu.VMEM((_GPW,), jnp.int32),
        pltpu.VMEM((_GPW,), jnp.int32),
        pltpu.VMEM((_GPW,), jnp.int32),
        pltpu.VMEM((_GPW * _P * 4,), jnp.int32),
        pltpu.VMEM((_GPW,), jnp.int32),
        pltpu.VMEM((_GPW,), jnp.int32),
        pltpu.VMEM((_GPW,), jnp.float32),
    ],
    compiler_params=pltpu.CompilerParams(needs_layout_passes=False),
    name="go_board_sc",
)


def kernel(board, board_history, positions, current_player, pass_count,
           move_count, roots, capture_groups):
    del pass_count
    board_flat = board.reshape(_B, _P)
    board_pad = jnp.pad(board_flat, ((0, 0), (0, _PPAD - _P)))
    roots_pad = jnp.pad(roots, ((0, 0), (0, _PPAD - _P)), constant_values=-1)
    rows = positions[:, 0]
    cols = positions[:, 1]
    cg1d = capture_groups.reshape(_B * _P * 4)
    hist_out = _tc_scat(move_count, board_flat, board_history)
    board_out = _sc_board(board_pad, roots_pad, rows, cols, current_player,
                          cg1d)
    new_board = board_out[:, :_P].reshape(_B, _BS, _BS)
    return new_board, hist_out


# speedup vs baseline: 5.2209x; 3.0431x over previous
import jax
import jax.numpy as jnp
from jax import lax
from jax.experimental import pallas as pl
from jax.experimental.pallas import tpu as pltpu
from jax.experimental.pallas import tpu_sc as plsc

_B = 512
_BS = 19
_P = _BS * _BS
_MAXM = _P
_EMPTY = -1.0
_LANES = 16
_NCH = _B // _LANES
_RTW = 40
_RTOFF_MAX = 328


_MB = 8
_NMB = (_MAXM + _MB - 1) // _MB


def _tc_hist_body(mv_ref, board_ref, hist_ref, out_ref):
    m0 = pl.program_id(0) * _MB
    m_ids = m0 + lax.broadcasted_iota(jnp.int32, (_MB, 1, 1), 0)
    mv = mv_ref[...].reshape(1, 1, _B)
    sel = m_ids == mv
    board = board_ref[...].reshape(1, _P, _B)
    out_ref[...] = jnp.where(sel, board, hist_ref[...])


_tc_hist = pl.pallas_call(
    _tc_hist_body,
    grid=(_NMB,),
    in_specs=[
        pl.BlockSpec((1, _B), lambda i: (0, 0)),
        pl.BlockSpec((_P, _B), lambda i: (0, 0)),
        pl.BlockSpec((_MB, _P, _B), lambda i: (i, 0, 0)),
    ],
    out_specs=pl.BlockSpec((_MB, _P, _B), lambda i: (i, 0, 0)),
    out_shape=jax.ShapeDtypeStruct((_MAXM, _P, _B), jnp.float32),
    compiler_params=pltpu.CompilerParams(
        dimension_semantics=("arbitrary",),
    ),
    name="go_hist_tc",
)


def _sc_board_body(board_h, roots_h, pos_h, ply_h, cg_h,
                   board_out,
                   b_v, rt_v, cg_v, g_v, r_v, c_v, ply_v):
    wid = lax.axis_index("s") * 2 + lax.axis_index("c")

    @pl.when(wid < _BS)
    def _():
        w = wid
        pltpu.sync_copy(pos_h.at[0], r_v)
        pltpu.sync_copy(pos_h.at[1], c_v)
        pltpu.sync_copy(ply_h, ply_v)
        pltpu.sync_copy(board_h.at[w], b_v)
        off = jnp.minimum((w * _BS) // 8 * 8, _RTOFF_MAX)
        off = pl.multiple_of(off, 8)
        local_r = w * _BS - off
        pltpu.sync_copy(roots_h.at[pl.ds(off, _RTW)], rt_v)

        iota = lax.iota(jnp.int32, _LANES)

        def rr_body(rr, carry):
            pltpu.sync_copy(cg_h.at[rr], cg_v)

            def ch_body(ch, c2):
                l0 = ch * _LANES
                lanes = l0 + iota
                rv = r_v[pl.ds(l0, _LANES)]
                cv = c_v[pl.ds(l0, _LANES)]
                rc = jnp.clip(rv, 0, _BS - 1)
                cc = jnp.clip(cv, 0, _BS - 1)
                hit = rc == rr
                for k in range(4):
                    kf = jnp.full((_LANES,), k, jnp.int32)
                    val = plsc.load_gather(cg_v, [cc, kf, lanes])
                    cur = g_v[k, pl.ds(l0, _LANES)]
                    g_v[k, pl.ds(l0, _LANES)] = jnp.where(hit, val, cur)
                return c2

            return lax.fori_loop(0, _NCH, ch_body, carry)

        lax.fori_loop(0, _BS, rr_body, jnp.int32(0))

        def ch2_body(ch, carry):
            l0 = ch * _LANES
            rv = r_v[pl.ds(l0, _LANES)]
            cv = c_v[pl.ds(l0, _LANES)]
            rc = jnp.clip(rv, 0, _BS - 1)
            cc = jnp.clip(cv, 0, _BS - 1)
            play = (rv >= 0) & (cv >= 0)
            ply = ply_v[pl.ds(l0, _LANES)].astype(jnp.float32)
            g0 = g_v[0, pl.ds(l0, _LANES)]
            g1 = g_v[1, pl.ds(l0, _LANES)]
            g2 = g_v[2, pl.ds(l0, _LANES)]
            g3 = g_v[3, pl.ds(l0, _LANES)]
            place_row = play & (rc == w)

            def c_body(c, c2):
                bvals = b_v[c, pl.ds(l0, _LANES)]
                rtv = rt_v[local_r + c, pl.ds(l0, _LANES)]
                v = jnp.where(place_row & (cc == c), ply, bvals)
                cap = (((rtv == g0) & (g0 >= 0)) | ((rtv == g1) & (g1 >= 0))
                       | ((rtv == g2) & (g2 >= 0)) | ((rtv == g3) & (g3 >= 0)))
                v = jnp.where(play & cap, jnp.float32(_EMPTY), v)
                b_v[c, pl.ds(l0, _LANES)] = v
                return c2

            return lax.fori_loop(0, _BS, c_body, carry)

        lax.fori_loop(0, _NCH, ch2_body, jnp.int32(0))

        pltpu.sync_copy(b_v, board_out.at[w])


_mesh = plsc.VectorSubcoreMesh(core_axis_name="c", subcore_axis_name="s")

_sc_board = pl.kernel(
    _sc_board_body,
    out_type=jax.ShapeDtypeStruct((_BS, _BS, _B), jnp.float32),
    mesh=_mesh,
    scratch_types=[
        pltpu.VMEM((_BS, _B), jnp.float32),
        pltpu.VMEM((_RTW, _B), jnp.int32),
        pltpu.VMEM((_BS, 4, _B), jnp.int32),
        pltpu.VMEM((4, _B), jnp.int32),
        pltpu.VMEM((_B,), jnp.int32),
        pltpu.VMEM((_B,), jnp.int32),
        pltpu.VMEM((_B,), jnp.int32),
    ],
    compiler_params=pltpu.CompilerParams(needs_layout_passes=False),
    name="go_board_sc",
)


def kernel(board, board_history, positions, current_player, pass_count,
           move_count, roots, capture_groups):
    del pass_count
    hist_t = board_history.transpose(1, 2, 0)
    board_t = board.transpose(1, 2, 0)
    board_r = board_t.reshape(_P, _B)
    roots_t = roots.transpose(1, 0)
    pos_t = positions.transpose(1, 0)
    cg_t = capture_groups.transpose(1, 2, 3, 0)

    hist_out_t = _tc_hist(move_count.reshape(1, _B), board_r, hist_t)
    board_out_t = _sc_board(board_t, roots_t, pos_t, current_player, cg_t)
    return board_out_t.transpose(2, 0, 1), hist_out_t.transpose(2, 0, 1)


# speedup vs baseline: 5.3134x; 1.0177x over previous
import jax
import jax.numpy as jnp
from jax import lax
from jax.experimental import pallas as pl
from jax.experimental.pallas import tpu as pltpu
from jax.experimental.pallas import tpu_sc as plsc

_B = 512
_BS = 19
_P = _BS * _BS
_MAXM = _P
_EMPTY = -1.0
_LANES = 16
_NCH = _B // _LANES
_RTW = 40
_RTOFF_MAX = 328


_MB = 16
_NMB = (_MAXM + _MB - 1) // _MB


def _tc_hist_body(mv_ref, board_ref, hist_ref, out_ref):
    m0 = pl.program_id(0) * _MB
    m_ids = m0 + lax.broadcasted_iota(jnp.int32, (_MB, 1, 1), 0)
    mv = mv_ref[...].reshape(1, 1, _B)
    sel = m_ids == mv
    board = board_ref[...].reshape(1, _P, _B)
    out_ref[...] = jnp.where(sel, board, hist_ref[...])


_tc_hist = pl.pallas_call(
    _tc_hist_body,
    grid=(_NMB,),
    in_specs=[
        pl.BlockSpec((1, _B), lambda i: (0, 0)),
        pl.BlockSpec((_P, _B), lambda i: (0, 0)),
        pl.BlockSpec((_MB, _P, _B), lambda i: (i, 0, 0)),
    ],
    out_specs=pl.BlockSpec((_MB, _P, _B), lambda i: (i, 0, 0)),
    out_shape=jax.ShapeDtypeStruct((_MAXM, _P, _B), jnp.float32),
    compiler_params=pltpu.CompilerParams(
        dimension_semantics=("arbitrary",),
    ),
    name="go_hist_tc",
)


def _sc_board_body(board_h, roots_h, pos_h, ply_h, cg_h,
                   board_out,
                   b_v, rt_v, cg_v, g_v, r_v, c_v, ply_v):
    wid = lax.axis_index("s") * 2 + lax.axis_index("c")

    @pl.when(wid < _BS)
    def _():
        w = wid
        pltpu.sync_copy(pos_h.at[0], r_v)
        pltpu.sync_copy(pos_h.at[1], c_v)
        pltpu.sync_copy(ply_h, ply_v)
        pltpu.sync_copy(board_h.at[w], b_v)
        off = jnp.minimum((w * _BS) // 8 * 8, _RTOFF_MAX)
        off = pl.multiple_of(off, 8)
        local_r = w * _BS - off
        pltpu.sync_copy(roots_h.at[pl.ds(off, _RTW)], rt_v)

        iota = lax.iota(jnp.int32, _LANES)

        def rr_body(rr, carry):
            pltpu.sync_copy(cg_h.at[rr], cg_v)

            def ch_body(ch, c2):
                l0 = ch * _LANES
                lanes = l0 + iota
                rv = r_v[pl.ds(l0, _LANES)]
                cv = c_v[pl.ds(l0, _LANES)]
                rc = jnp.clip(rv, 0, _BS - 1)
                cc = jnp.clip(cv, 0, _BS - 1)
                hit = rc == rr
                for k in range(4):
                    kf = jnp.full((_LANES,), k, jnp.int32)
                    val = plsc.load_gather(cg_v, [cc, kf, lanes])
                    cur = g_v[k, pl.ds(l0, _LANES)]
                    g_v[k, pl.ds(l0, _LANES)] = jnp.where(hit, val, cur)
                return c2

            return lax.fori_loop(0, _NCH, ch_body, carry)

        lax.fori_loop(0, _BS, rr_body, jnp.int32(0))

        def ch2_body(ch, carry):
            l0 = ch * _LANES
            rv = r_v[pl.ds(l0, _LANES)]
            cv = c_v[pl.ds(l0, _LANES)]
            rc = jnp.clip(rv, 0, _BS - 1)
            cc = jnp.clip(cv, 0, _BS - 1)
            play = (rv >= 0) & (cv >= 0)
            ply = ply_v[pl.ds(l0, _LANES)].astype(jnp.float32)
            g0 = g_v[0, pl.ds(l0, _LANES)]
            g1 = g_v[1, pl.ds(l0, _LANES)]
            g2 = g_v[2, pl.ds(l0, _LANES)]
            g3 = g_v[3, pl.ds(l0, _LANES)]
            place_row = play & (rc == w)

            def c_body(c, c2):
                bvals = b_v[c, pl.ds(l0, _LANES)]
                rtv = rt_v[local_r + c, pl.ds(l0, _LANES)]
                v = jnp.where(place_row & (cc == c), ply, bvals)
                cap = (((rtv == g0) & (g0 >= 0)) | ((rtv == g1) & (g1 >= 0))
                       | ((rtv == g2) & (g2 >= 0)) | ((rtv == g3) & (g3 >= 0)))
                v = jnp.where(play & cap, jnp.float32(_EMPTY), v)
                b_v[c, pl.ds(l0, _LANES)] = v
                return c2

            return lax.fori_loop(0, _BS, c_body, carry)

        lax.fori_loop(0, _NCH, ch2_body, jnp.int32(0))

        pltpu.sync_copy(b_v, board_out.at[w])


_mesh = plsc.VectorSubcoreMesh(core_axis_name="c", subcore_axis_name="s")

_sc_board = pl.kernel(
    _sc_board_body,
    out_type=jax.ShapeDtypeStruct((_BS, _BS, _B), jnp.float32),
    mesh=_mesh,
    scratch_types=[
        pltpu.VMEM((_BS, _B), jnp.float32),
        pltpu.VMEM((_RTW, _B), jnp.int32),
        pltpu.VMEM((_BS, 4, _B), jnp.int32),
        pltpu.VMEM((4, _B), jnp.int32),
        pltpu.VMEM((_B,), jnp.int32),
        pltpu.VMEM((_B,), jnp.int32),
        pltpu.VMEM((_B,), jnp.int32),
    ],
    compiler_params=pltpu.CompilerParams(needs_layout_passes=False),
    name="go_board_sc",
)


def kernel(board, board_history, positions, current_player, pass_count,
           move_count, roots, capture_groups):
    del pass_count
    hist_t = board_history.transpose(1, 2, 0)
    board_t = board.transpose(1, 2, 0)
    board_r = board_t.reshape(_P, _B)
    roots_t = roots.transpose(1, 0)
    pos_t = positions.transpose(1, 0)
    cg_t = capture_groups.transpose(1, 2, 3, 0)

    hist_out_t = _tc_hist(move_count.reshape(1, _B), board_r, hist_t)
    board_out_t = _sc_board(board_t, roots_t, pos_t, current_player, cg_t)
    return board_out_t.transpose(2, 0, 1), hist_out_t.transpose(2, 0, 1)


# speedup vs baseline: 5.3198x; 1.0012x over previous
import jax
import jax.numpy as jnp
from jax import lax
from jax.experimental import pallas as pl
from jax.experimental.pallas import tpu as pltpu
from jax.experimental.pallas import tpu_sc as plsc

_B = 512
_BS = 19
_P = _BS * _BS
_MAXM = _P
_EMPTY = -1.0
_LANES = 16
_NCH = _B // _LANES
_RTW = 40
_RTOFF_MAX = 328


_MB = 18
_NMB = (_MAXM + _MB - 1) // _MB


def _tc_hist_body(mv_ref, board_ref, hist_ref, out_ref):
    m0 = pl.program_id(0) * _MB
    m_ids = m0 + lax.broadcasted_iota(jnp.int32, (_MB, 1, 1), 0)
    mv = mv_ref[...].reshape(1, 1, _B)
    sel = m_ids == mv
    board = board_ref[...].reshape(1, _P, _B)
    out_ref[...] = jnp.where(sel, board, hist_ref[...])


_tc_hist = pl.pallas_call(
    _tc_hist_body,
    grid=(_NMB,),
    in_specs=[
        pl.BlockSpec((1, _B), lambda i: (0, 0)),
        pl.BlockSpec((_P, _B), lambda i: (0, 0)),
        pl.BlockSpec((_MB, _P, _B), lambda i: (i, 0, 0)),
    ],
    out_specs=pl.BlockSpec((_MB, _P, _B), lambda i: (i, 0, 0)),
    out_shape=jax.ShapeDtypeStruct((_MAXM, _P, _B), jnp.float32),
    compiler_params=pltpu.CompilerParams(
        dimension_semantics=("arbitrary",),
    ),
    name="go_hist_tc",
)


def _sc_board_body(board_h, roots_h, pos_h, ply_h, cg_h,
                   board_out,
                   b_v, rt_v, cg_v, g_v, r_v, c_v, ply_v):
    wid = lax.axis_index("s") * 2 + lax.axis_index("c")

    @pl.when(wid < _BS)
    def _():
        w = wid
        pltpu.sync_copy(pos_h.at[0], r_v)
        pltpu.sync_copy(pos_h.at[1], c_v)
        pltpu.sync_copy(ply_h, ply_v)
        pltpu.sync_copy(board_h.at[w], b_v)
        off = jnp.minimum((w * _BS) // 8 * 8, _RTOFF_MAX)
        off = pl.multiple_of(off, 8)
        local_r = w * _BS - off
        pltpu.sync_copy(roots_h.at[pl.ds(off, _RTW)], rt_v)

        iota = lax.iota(jnp.int32, _LANES)

        def rr_body(rr, carry):
            pltpu.sync_copy(cg_h.at[rr], cg_v)

            def ch_body(ch, c2):
                l0 = ch * _LANES
                lanes = l0 + iota
                rv = r_v[pl.ds(l0, _LANES)]
                cv = c_v[pl.ds(l0, _LANES)]
                rc = jnp.clip(rv, 0, _BS - 1)
                cc = jnp.clip(cv, 0, _BS - 1)
                hit = rc == rr
                for k in range(4):
                    kf = jnp.full((_LANES,), k, jnp.int32)
                    val = plsc.load_gather(cg_v, [cc, kf, lanes])
                    cur = g_v[k, pl.ds(l0, _LANES)]
                    g_v[k, pl.ds(l0, _LANES)] = jnp.where(hit, val, cur)
                return c2

            return lax.fori_loop(0, _NCH, ch_body, carry)

        lax.fori_loop(0, _BS, rr_body, jnp.int32(0))

        def ch2_body(ch, carry):
            l0 = ch * _LANES
            rv = r_v[pl.ds(l0, _LANES)]
            cv = c_v[pl.ds(l0, _LANES)]
            rc = jnp.clip(rv, 0, _BS - 1)
            cc = jnp.clip(cv, 0, _BS - 1)
            play = (rv >= 0) & (cv >= 0)
            ply = ply_v[pl.ds(l0, _LANES)].astype(jnp.float32)
            g0 = g_v[0, pl.ds(l0, _LANES)]
            g1 = g_v[1, pl.ds(l0, _LANES)]
            g2 = g_v[2, pl.ds(l0, _LANES)]
            g3 = g_v[3, pl.ds(l0, _LANES)]
            place_row = play & (rc == w)

            def c_body(c, c2):
                bvals = b_v[c, pl.ds(l0, _LANES)]
                rtv = rt_v[local_r + c, pl.ds(l0, _LANES)]
                v = jnp.where(place_row & (cc == c), ply, bvals)
                cap = (((rtv == g0) & (g0 >= 0)) | ((rtv == g1) & (g1 >= 0))
                       | ((rtv == g2) & (g2 >= 0)) | ((rtv == g3) & (g3 >= 0)))
                v = jnp.where(play & cap, jnp.float32(_EMPTY), v)
                b_v[c, pl.ds(l0, _LANES)] = v
                return c2

            return lax.fori_loop(0, _BS, c_body, carry)

        lax.fori_loop(0, _NCH, ch2_body, jnp.int32(0))

        pltpu.sync_copy(b_v, board_out.at[w])


_mesh = plsc.VectorSubcoreMesh(core_axis_name="c", subcore_axis_name="s")

_sc_board = pl.kernel(
    _sc_board_body,
    out_type=jax.ShapeDtypeStruct((_BS, _BS, _B), jnp.float32),
    mesh=_mesh,
    scratch_types=[
        pltpu.VMEM((_BS, _B), jnp.float32),
        pltpu.VMEM((_RTW, _B), jnp.int32),
        pltpu.VMEM((_BS, 4, _B), jnp.int32),
        pltpu.VMEM((4, _B), jnp.int32),
        pltpu.VMEM((_B,), jnp.int32),
        pltpu.VMEM((_B,), jnp.int32),
        pltpu.VMEM((_B,), jnp.int32),
    ],
    compiler_params=pltpu.CompilerParams(needs_layout_passes=False),
    name="go_board_sc",
)


def kernel(board, board_history, positions, current_player, pass_count,
           move_count, roots, capture_groups):
    del pass_count
    hist_t = board_history.transpose(1, 2, 0)
    board_t = board.transpose(1, 2, 0)
    board_r = board_t.reshape(_P, _B)
    roots_t = roots.transpose(1, 0)
    pos_t = positions.transpose(1, 0)
    cg_t = capture_groups.transpose(1, 2, 3, 0)

    hist_out_t = _tc_hist(move_count.reshape(1, _B), board_r, hist_t)
    board_out_t = _sc_board(board_t, roots_t, pos_t, current_player, cg_t)
    return board_out_t.transpose(2, 0, 1), hist_out_t.transpose(2, 0, 1)


# speedup vs baseline: 5.3210x; 1.0002x over previous
import jax
import jax.numpy as jnp
from jax import lax
from jax.experimental import pallas as pl
from jax.experimental.pallas import tpu as pltpu
from jax.experimental.pallas import tpu_sc as plsc

_B = 512
_BS = 19
_P = _BS * _BS
_MAXM = _P
_EMPTY = -1.0
_LANES = 16
_NCH = _B // _LANES
_RTW = 40
_RTOFF_MAX = 328


_MB = 19
_NMB = (_MAXM + _MB - 1) // _MB


def _tc_hist_body(mv_ref, board_ref, hist_ref, out_ref):
    m0 = pl.program_id(0) * _MB
    m_ids = m0 + lax.broadcasted_iota(jnp.int32, (_MB, 1, 1), 0)
    mv = mv_ref[...].reshape(1, 1, _B)
    sel = m_ids == mv
    board = board_ref[...].reshape(1, _P, _B)
    out_ref[...] = jnp.where(sel, board, hist_ref[...])


_tc_hist = pl.pallas_call(
    _tc_hist_body,
    grid=(_NMB,),
    in_specs=[
        pl.BlockSpec((1, _B), lambda i: (0, 0)),
        pl.BlockSpec((_P, _B), lambda i: (0, 0)),
        pl.BlockSpec((_MB, _P, _B), lambda i: (i, 0, 0)),
    ],
    out_specs=pl.BlockSpec((_MB, _P, _B), lambda i: (i, 0, 0)),
    out_shape=jax.ShapeDtypeStruct((_MAXM, _P, _B), jnp.float32),
    compiler_params=pltpu.CompilerParams(
        dimension_semantics=("arbitrary",),
    ),
    name="go_hist_tc",
)


def _sc_board_body(board_h, roots_h, pos_h, ply_h, cg_h,
                   board_out,
                   b_v, rt_v, cg_v, g_v, r_v, c_v, ply_v):
    wid = lax.axis_index("s") * 2 + lax.axis_index("c")

    @pl.when(wid < _BS)
    def _():
        w = wid
        pltpu.sync_copy(pos_h.at[0], r_v)
        pltpu.sync_copy(pos_h.at[1], c_v)
        pltpu.sync_copy(ply_h, ply_v)
        pltpu.sync_copy(board_h.at[w], b_v)
        off = jnp.minimum((w * _BS) // 8 * 8, _RTOFF_MAX)
        off = pl.multiple_of(off, 8)
        local_r = w * _BS - off
        pltpu.sync_copy(roots_h.at[pl.ds(off, _RTW)], rt_v)

        iota = lax.iota(jnp.int32, _LANES)

        def rr_body(rr, carry):
            pltpu.sync_copy(cg_h.at[rr], cg_v)

            def ch_body(ch, c2):
                l0 = ch * _LANES
                lanes = l0 + iota
                rv = r_v[pl.ds(l0, _LANES)]
                cv = c_v[pl.ds(l0, _LANES)]
                rc = jnp.clip(rv, 0, _BS - 1)
                cc = jnp.clip(cv, 0, _BS - 1)
                hit = rc == rr
                for k in range(4):
                    kf = jnp.full((_LANES,), k, jnp.int32)
                    val = plsc.load_gather(cg_v, [cc, kf, lanes])
                    cur = g_v[k, pl.ds(l0, _LANES)]
                    g_v[k, pl.ds(l0, _LANES)] = jnp.where(hit, val, cur)
                return c2

            return lax.fori_loop(0, _NCH, ch_body, carry)

        lax.fori_loop(0, _BS, rr_body, jnp.int32(0))

        def ch2_body(ch, carry):
            l0 = ch * _LANES
            rv = r_v[pl.ds(l0, _LANES)]
            cv = c_v[pl.ds(l0, _LANES)]
            rc = jnp.clip(rv, 0, _BS - 1)
            cc = jnp.clip(cv, 0, _BS - 1)
            play = (rv >= 0) & (cv >= 0)
            ply = ply_v[pl.ds(l0, _LANES)].astype(jnp.float32)
            g0 = g_v[0, pl.ds(l0, _LANES)]
            g1 = g_v[1, pl.ds(l0, _LANES)]
            g2 = g_v[2, pl.ds(l0, _LANES)]
            g3 = g_v[3, pl.ds(l0, _LANES)]
            place_row = play & (rc == w)

            def c_body(c, c2):
                bvals = b_v[c, pl.ds(l0, _LANES)]
                rtv = rt_v[local_r + c, pl.ds(l0, _LANES)]
                v = jnp.where(place_row & (cc == c), ply, bvals)
                cap = (((rtv == g0) & (g0 >= 0)) | ((rtv == g1) & (g1 >= 0))
                       | ((rtv == g2) & (g2 >= 0)) | ((rtv == g3) & (g3 >= 0)))
                v = jnp.where(play & cap, jnp.float32(_EMPTY), v)
                b_v[c, pl.ds(l0, _LANES)] = v
                return c2

            return lax.fori_loop(0, _BS, c_body, carry)

        lax.fori_loop(0, _NCH, ch2_body, jnp.int32(0))

        pltpu.sync_copy(b_v, board_out.at[w])


_mesh = plsc.VectorSubcoreMesh(core_axis_name="c", subcore_axis_name="s")

_sc_board = pl.kernel(
    _sc_board_body,
    out_type=jax.ShapeDtypeStruct((_BS, _BS, _B), jnp.float32),
    mesh=_mesh,
    scratch_types=[
        pltpu.VMEM((_BS, _B), jnp.float32),
        pltpu.VMEM((_RTW, _B), jnp.int32),
        pltpu.VMEM((_BS, 4, _B), jnp.int32),
        pltpu.VMEM((4, _B), jnp.int32),
        pltpu.VMEM((_B,), jnp.int32),
        pltpu.VMEM((_B,), jnp.int32),
        pltpu.VMEM((_B,), jnp.int32),
    ],
    compiler_params=pltpu.CompilerParams(needs_layout_passes=False),
    name="go_board_sc",
)


def kernel(board, board_history, positions, current_player, pass_count,
           move_count, roots, capture_groups):
    del pass_count
    hist_t = board_history.transpose(1, 2, 0)
    board_t = board.transpose(1, 2, 0)
    board_r = board_t.reshape(_P, _B)
    roots_t = roots.transpose(1, 0)
    pos_t = positions.transpose(1, 0)
    cg_t = capture_groups.transpose(1, 2, 3, 0)

    hist_out_t = _tc_hist(move_count.reshape(1, _B), board_r, hist_t)
    board_out_t = _sc_board(board_t, roots_t, pos_t, current_player, cg_t)
    return board_out_t.transpose(2, 0, 1), hist_out_t.transpose(2, 0, 1)
